# baseline (speedup 1.0000x reference)
import functools

import jax
import jax.numpy as jnp
from jax import lax
from jax.experimental import pallas as pl
from jax.experimental.pallas import tpu as pltpu
from jax.experimental.pallas import tpu_sc as plsc

ROWS, COLS = 16384, 200
NC, NS, L = 2, 16, 16
NW = NC * NS
COLS_W = ROWS // NW
CH = 128
NCHUNK = COLS_W // CH


@functools.cache
def _make_emb_sc():
    mesh = plsc.VectorSubcoreMesh(
        core_axis_name="c", subcore_axis_name="s",
        num_cores=NC, num_subcores=NS)

    @functools.partial(
        pl.kernel,
        out_type=jax.ShapeDtypeStruct((COLS, ROWS // 128, 128), jnp.float32),
        mesh=mesh,
        scratch_types=[
            pltpu.VMEM((4, 1), jnp.float32),
            pltpu.VMEM((L,), jnp.float32),
            pltpu.VMEM((COLS, CH), jnp.int32),
            pltpu.VMEM((COLS, CH), jnp.int32),
            pltpu.VMEM((COLS, CH // 128, 128), jnp.float32),
            pltpu.VMEM((COLS, CH // 128, 128), jnp.float32),
            pltpu.SemaphoreType.DMA,
            pltpu.SemaphoreType.DMA,
            pltpu.SemaphoreType.DMA,
            pltpu.SemaphoreType.DMA,
        ],
        compiler_params=pltpu.CompilerParams(
            needs_layout_passes=False, use_tc_tiling_on_sc=True),
    )
    def emb_sc(x_hbm, tbl_hbm, out_hbm, tbl2_v, tbl_v, xb0, xb1, ob0, ob1,
               isem0, isem1, osem0, osem1):
        wid = lax.axis_index("s") * NC + lax.axis_index("c")
        base = wid * COLS_W
        pltpu.sync_copy(tbl_hbm, tbl2_v)
        lane = lax.iota(jnp.int32, L)
        tbl_v[...] = plsc.load_gather(tbl2_v, [lane & 3, lane * 0])
        xb, ob = (xb0, xb1), (ob0, ob1)
        isem, osem = (isem0, isem1), (osem0, osem1)
        in_cp = [None, None]
        out_cp = [None, None]
        in_cp[0] = pltpu.async_copy(x_hbm.at[:, pl.ds(base, CH)], xb[0], isem[0])
        for c in range(NCHUNK):
            b = c & 1
            if c + 1 < NCHUNK:
                in_cp[1 - b] = pltpu.async_copy(
                    x_hbm.at[:, pl.ds(base + (c + 1) * CH, CH)],
                    xb[1 - b], isem[1 - b])
            in_cp[b].wait()
            if out_cp[b] is not None:
                out_cp[b].wait()

            def body(j, xr=xb[b], orf=ob[b]):
                for k in range(CH // L):
                    idx = xr[j, pl.ds(k * L, L)]
                    orf[j, k // 8, pl.ds((k % 8) * L, L)] = (
                        plsc.load_gather(tbl_v, [idx]))

            plsc.parallel_loop(0, COLS, 1, unroll=4)(body)
            out_cp[b] = pltpu.async_copy(
                ob[b],
                out_hbm.at[:, pl.ds((base + c * CH) // 128, CH // 128), :],
                osem[b])
        out_cp[(NCHUNK - 1) & 1].wait()
        if NCHUNK > 1:
            out_cp[NCHUNK & 1].wait()

    return emb_sc


BLK = ROWS // 16


def _add_body(h_ref, o_ref):
    o_ref[...] = h_ref[...] + jnp.float32(COLS)


def _add_tc(ht):
    return pl.pallas_call(
        _add_body,
        out_shape=jax.ShapeDtypeStruct((COLS, ROWS), jnp.float32),
        grid=(ROWS // BLK,),
        in_specs=[pl.BlockSpec((COLS, BLK), lambda i: (0, i))],
        out_specs=pl.BlockSpec((COLS, BLK), lambda i: (0, i)),
    )(ht)


def kernel(x, h, table):
    emb_t3 = _make_emb_sc()(x.T, table)
    out2 = _add_tc(h.T).T
    emb = emb_t3.transpose(1, 2, 0).reshape(ROWS, COLS, 1)
    return (emb, out2)


# speedup vs baseline: 1.0225x; 1.0225x over previous
import functools

import jax
import jax.numpy as jnp
from jax import lax
from jax.experimental import pallas as pl
from jax.experimental.pallas import tpu as pltpu
from jax.experimental.pallas import tpu_sc as plsc

ROWS, COLS = 16384, 200
NC, NS, L = 2, 16, 16
NW = NC * NS
COLS_W = ROWS // NW
CH = 128
NCHUNK = COLS_W // CH


@functools.cache
def _make_emb_sc():
    mesh = plsc.VectorSubcoreMesh(
        core_axis_name="c", subcore_axis_name="s",
        num_cores=NC, num_subcores=NS)

    @functools.partial(
        pl.kernel,
        out_type=jax.ShapeDtypeStruct((COLS, ROWS // 128, 128), jnp.float32),
        mesh=mesh,
        scratch_types=[
            pltpu.VMEM((4, 1), jnp.float32),
            pltpu.VMEM((L,), jnp.float32),
            pltpu.VMEM((COLS, CH), jnp.int32),
            pltpu.VMEM((COLS, CH), jnp.int32),
            pltpu.VMEM((COLS, CH), jnp.int32),
            pltpu.VMEM((COLS, CH // 128, 128), jnp.float32),
            pltpu.VMEM((COLS, CH // 128, 128), jnp.float32),
            pltpu.SemaphoreType.DMA,
            pltpu.SemaphoreType.DMA,
            pltpu.SemaphoreType.DMA,
            pltpu.SemaphoreType.DMA,
            pltpu.SemaphoreType.DMA,
        ],
        compiler_params=pltpu.CompilerParams(
            needs_layout_passes=False, use_tc_tiling_on_sc=True),
    )
    def emb_sc(x_hbm, tbl_hbm, out_hbm, tbl2_v, tbl_v, xb0, xb1, xb2,
               ob0, ob1, isem0, isem1, isem2, osem0, osem1):
        wid = lax.axis_index("s") * NC + lax.axis_index("c")
        base = wid * COLS_W
        pltpu.sync_copy(tbl_hbm, tbl2_v)
        lane = lax.iota(jnp.int32, L)
        tbl_v[...] = plsc.load_gather(tbl2_v, [lane & 3, lane * 0])
        xb, ob = (xb0, xb1, xb2), (ob0, ob1)
        isem, osem = (isem0, isem1, isem2), (osem0, osem1)
        in_cp = [None, None, None]
        out_cp = [None, None]
        for p in range(2):
            in_cp[p] = pltpu.async_copy(
                x_hbm.at[:, pl.ds(base + p * CH, CH)], xb[p], isem[p])
        for c in range(NCHUNK):
            bi = c % 3
            b = c & 1
            if c + 2 < NCHUNK:
                nb = (c + 2) % 3
                in_cp[nb] = pltpu.async_copy(
                    x_hbm.at[:, pl.ds(base + (c + 2) * CH, CH)],
                    xb[nb], isem[nb])
            in_cp[bi].wait()
            if out_cp[b] is not None:
                out_cp[b].wait()

            def body(j, xr=xb[bi], orf=ob[b]):
                for k in range(CH // L):
                    idx = xr[j, pl.ds(k * L, L)]
                    orf[j, k // 8, pl.ds((k % 8) * L, L)] = (
                        plsc.load_gather(tbl_v, [idx]))

            plsc.parallel_loop(0, COLS, 1, unroll=4)(body)
            out_cp[b] = pltpu.async_copy(
                ob[b],
                out_hbm.at[:, pl.ds((base + c * CH) // 128, CH // 128), :],
                osem[b])
        out_cp[(NCHUNK - 1) & 1].wait()
        if NCHUNK > 1:
            out_cp[NCHUNK & 1].wait()

    return emb_sc


BLK = ROWS // 16


def _add_body(h_ref, o_ref):
    o_ref[...] = h_ref[...] + jnp.float32(COLS)


def _add_tc(ht):
    return pl.pallas_call(
        _add_body,
        out_shape=jax.ShapeDtypeStruct((COLS, ROWS), jnp.float32),
        grid=(ROWS // BLK,),
        in_specs=[pl.BlockSpec((COLS, BLK), lambda i: (0, i))],
        out_specs=pl.BlockSpec((COLS, BLK), lambda i: (0, i)),
    )(ht)


def kernel(x, h, table):
    emb_t3 = _make_emb_sc()(x.T, table)
    out2 = _add_tc(h.T).T
    emb = emb_t3.transpose(1, 2, 0).reshape(ROWS, COLS, 1)
    return (emb, out2)


# speedup vs baseline: 1.0492x; 1.0261x over previous
import functools

import jax
import jax.numpy as jnp
from jax import lax
from jax.experimental import pallas as pl
from jax.experimental.pallas import tpu as pltpu
from jax.experimental.pallas import tpu_sc as plsc

ROWS, COLS = 16384, 200
NC, NS, L = 2, 16, 16
NW = NC * NS
COLS_W = ROWS // NW
CH = 128
NCHUNK = COLS_W // CH


@functools.cache
def _make_emb_sc():
    mesh = plsc.VectorSubcoreMesh(
        core_axis_name="c", subcore_axis_name="s",
        num_cores=NC, num_subcores=NS)

    @functools.partial(
        pl.kernel,
        out_type=jax.ShapeDtypeStruct((COLS, ROWS // 128, 128), jnp.float32),
        mesh=mesh,
        scratch_types=[
            pltpu.VMEM((4, 1), jnp.float32),
            pltpu.VMEM((L,), jnp.float32),
            pltpu.VMEM((COLS, CH), jnp.int32),
            pltpu.VMEM((COLS, CH), jnp.int32),
            pltpu.VMEM((COLS, CH), jnp.int32),
            pltpu.VMEM((COLS, CH // 128, 128), jnp.float32),
            pltpu.VMEM((COLS, CH // 128, 128), jnp.float32),
            pltpu.SemaphoreType.DMA,
            pltpu.SemaphoreType.DMA,
            pltpu.SemaphoreType.DMA,
            pltpu.SemaphoreType.DMA,
            pltpu.SemaphoreType.DMA,
        ],
        compiler_params=pltpu.CompilerParams(
            needs_layout_passes=False, use_tc_tiling_on_sc=True),
    )
    def emb_sc(x_hbm, tbl_hbm, out_hbm, tbl2_v, tbl_v, xb0, xb1, xb2,
               ob0, ob1, isem0, isem1, isem2, osem0, osem1):
        wid = lax.axis_index("s") * NC + lax.axis_index("c")
        base = wid * COLS_W
        pltpu.sync_copy(tbl_hbm, tbl2_v)
        lane = lax.iota(jnp.int32, L)
        tbl_v[...] = plsc.load_gather(tbl2_v, [lane & 3, lane * 0])
        xb, ob = (xb0, xb1, xb2), (ob0, ob1)
        isem, osem = (isem0, isem1, isem2), (osem0, osem1)
        in_cp = [None, None, None]
        out_cp = [None, None]
        for p in range(2):
            in_cp[p] = pltpu.async_copy(
                x_hbm.at[:, pl.ds(base + p * CH, CH)], xb[p], isem[p])
        for c in range(NCHUNK):
            bi = c % 3
            b = c & 1
            if c + 2 < NCHUNK:
                nb = (c + 2) % 3
                in_cp[nb] = pltpu.async_copy(
                    x_hbm.at[:, pl.ds(base + (c + 2) * CH, CH)],
                    xb[nb], isem[nb])
            in_cp[bi].wait()
            if out_cp[b] is not None:
                out_cp[b].wait()

            def body(j, xr=xb[bi], orf=ob[b]):
                for k in range(CH // L):
                    idx = xr[j, pl.ds(k * L, L)]
                    orf[j, k // 8, pl.ds((k % 8) * L, L)] = (
                        plsc.load_gather(tbl_v, [idx]))

            plsc.parallel_loop(0, COLS, 1, unroll=1)(body)
            out_cp[b] = pltpu.async_copy(
                ob[b],
                out_hbm.at[:, pl.ds((base + c * CH) // 128, CH // 128), :],
                osem[b])
        out_cp[(NCHUNK - 1) & 1].wait()
        if NCHUNK > 1:
            out_cp[NCHUNK & 1].wait()

    return emb_sc


BLK = ROWS // 16


def _add_body(h_ref, o_ref):
    o_ref[...] = h_ref[...] + jnp.float32(COLS)


def _add_tc(ht):
    return pl.pallas_call(
        _add_body,
        out_shape=jax.ShapeDtypeStruct((COLS, ROWS), jnp.float32),
        grid=(ROWS // BLK,),
        in_specs=[pl.BlockSpec((COLS, BLK), lambda i: (0, i))],
        out_specs=pl.BlockSpec((COLS, BLK), lambda i: (0, i)),
    )(ht)


def kernel(x, h, table):
    emb_t3 = _make_emb_sc()(x.T, table)
    out2 = _add_tc(h.T).T
    emb = emb_t3.transpose(1, 2, 0).reshape(ROWS, COLS, 1)
    return (emb, out2)
